# baseline (speedup 1.0000x reference)
import functools

import jax
import jax.numpy as jnp
from jax import lax
from jax.experimental import pallas as pl
from jax.experimental.pallas import tpu as pltpu
from jax.experimental.pallas import tpu_sc as plsc

VOCAB = 1_000_000
DIM = 32
XROWS = 16384
XCOLS = 50
NUM_IDX = XROWS * XCOLS
PHYS_ROWS = VOCAB // 4
PACK = 128 // DIM
NUM_WORKERS = 32
IPW = NUM_IDX // NUM_WORKERS
B = 256
NB = IPW // B
NHALF = NB // 2
PB = B // PACK
L = 16


def _body(x_hbm, emb_hbm, out_hbm, idx_v, pidx_v, offs_v, rows_v, outb_v,
          gsem_a, gsem_b, osem_a, osem_b):
    c = lax.axis_index("c")
    s = lax.axis_index("s")
    wid = s * 2 + c
    ibase = pl.multiple_of(wid * IPW, IPW)
    pltpu.sync_copy(x_hbm.at[pl.ds(ibase, IPW)], idx_v)

    def prep(batch, half):
        for k in range(B // L):
            v = idx_v[pl.ds(batch * B + k * L, L)]
            pidx_v[pl.ds(half * B + k * L, L)] = lax.shift_right_logical(v, 2)
            offs_v[pl.ds(half * B + k * L, L)] = lax.shift_left(
                lax.bitwise_and(v, 3), 5
            )

    def fire(half, sem):
        descs = []
        for k in range(B // 128):
            descs.append(
                pltpu.async_copy(
                    emb_hbm.at[pidx_v.at[pl.ds(half * B + k * 128, 128)]],
                    rows_v.at[pl.ds(half * B + k * 128, 128)],
                    sem,
                )
            )
        return descs

    def extract(half):
        def m_body(m, carry):
            i0 = half * B + m * PACK
            offq = offs_v[pl.ds(i0, L)]
            for q in range(PACK):
                i = i0 + q
                off = offq[q]
                v0 = rows_v[i, pl.ds(off, L)]
                v1 = rows_v[i, pl.ds(off + L, L)]
                outb_v[half * PB + m, pl.ds(q * DIM, L)] = v0
                outb_v[half * PB + m, pl.ds(q * DIM + L, L)] = v1
            return carry

        lax.fori_loop(0, PB, m_body, 0, unroll=2)

    def out_copy(batch, half, sem):
        row0 = pl.multiple_of((ibase + batch * B) // PACK, PB)
        return pltpu.async_copy(
            outb_v.at[pl.ds(half * PB, PB)],
            out_hbm.at[pl.ds(row0, PB)],
            sem,
        )

    def outer(t, carry):
        prep(2 * t, 0)
        ga = fire(0, gsem_a)
        prep(2 * t + 1, 1)
        gb = fire(1, gsem_b)
        for d in ga:
            d.wait()
        extract(0)
        oa = out_copy(2 * t, 0, osem_a)
        for d in gb:
            d.wait()
        extract(1)
        ob = out_copy(2 * t + 1, 1, osem_b)
        oa.wait()
        ob.wait()
        return carry

    lax.fori_loop(0, NHALF, outer, 0)


@jax.jit
def _gather(xflat, emb4):
    f = functools.partial(
        pl.kernel,
        out_type=jax.ShapeDtypeStruct((NUM_IDX // PACK, 128), jnp.float32),
        mesh=plsc.VectorSubcoreMesh(core_axis_name="c", subcore_axis_name="s"),
        scratch_types=[
            pltpu.VMEM((IPW,), jnp.int32),
            pltpu.VMEM((2 * B,), jnp.int32),
            pltpu.VMEM((2 * B + L,), jnp.int32),
            pltpu.VMEM((2 * B, 128), jnp.float32),
            pltpu.VMEM((2 * PB, 128), jnp.float32),
            pltpu.SemaphoreType.DMA,
            pltpu.SemaphoreType.DMA,
            pltpu.SemaphoreType.DMA,
            pltpu.SemaphoreType.DMA,
        ],
    )(_body)
    return f(xflat, emb4)


def kernel(x, embeddings):
    xflat = x.reshape(NUM_IDX)
    emb4 = embeddings.reshape(PHYS_ROWS, 128)
    out = _gather(xflat, emb4)
    return out.reshape(XROWS, XCOLS, DIM)


# speedup vs baseline: 1.3096x; 1.3096x over previous
import functools

import jax
import jax.numpy as jnp
from jax import lax
from jax.experimental import pallas as pl
from jax.experimental.pallas import tpu as pltpu
from jax.experimental.pallas import tpu_sc as plsc

VOCAB = 1_000_000
DIM = 32
XROWS = 16384
XCOLS = 50
NUM_WORKERS = 32
XR_PER_WORKER = XROWS // NUM_WORKERS
XR_PER_BATCH = 16
BATCH_ROWS = XR_PER_BATCH * XCOLS
T = XR_PER_WORKER // XR_PER_BATCH
THALF = T // 2


def _body(x_hbm, emb_hbm, out_hbm, idx_v, rows_v, gsem_a, gsem_b, osem_a, osem_b):
    c = lax.axis_index("c")
    s = lax.axis_index("s")
    wid = s * 2 + c
    xrbase = wid * XR_PER_WORKER
    pltpu.sync_copy(x_hbm.at[pl.ds(xrbase, XR_PER_WORKER)], idx_v)

    def fire(batch, half, sem):
        descs = []
        for j in range(XR_PER_BATCH):
            xr = batch * XR_PER_BATCH + j
            descs.append(
                pltpu.async_copy(
                    emb_hbm.at[idx_v.at[xr]],
                    rows_v.at[pl.ds(half * BATCH_ROWS + j * XCOLS, XCOLS)],
                    sem,
                )
            )
        return descs

    def out_copy(batch, half, sem):
        descs = []
        xr0 = xrbase + batch * XR_PER_BATCH
        for j in range(XR_PER_BATCH):
            descs.append(
                pltpu.async_copy(
                    rows_v.at[pl.ds(half * BATCH_ROWS + j * XCOLS, XCOLS)],
                    out_hbm.at[xr0 + j],
                    sem,
                )
            )
        return descs

    def outer(t, carry):
        ga = fire(2 * t, 0, gsem_a)
        gb = fire(2 * t + 1, 1, gsem_b)
        for d in ga:
            d.wait()
        oa = out_copy(2 * t, 0, osem_a)
        for d in gb:
            d.wait()
        ob = out_copy(2 * t + 1, 1, osem_b)
        for d in oa:
            d.wait()
        for d in ob:
            d.wait()
        return carry

    lax.fori_loop(0, THALF, outer, 0)


@jax.jit
def _gather(x, embeddings):
    f = functools.partial(
        pl.kernel,
        out_type=jax.ShapeDtypeStruct((XROWS, XCOLS, DIM), jnp.float32),
        mesh=plsc.VectorSubcoreMesh(core_axis_name="c", subcore_axis_name="s"),
        scratch_types=[
            pltpu.VMEM((XR_PER_WORKER, XCOLS), jnp.int32),
            pltpu.VMEM((2 * BATCH_ROWS, DIM), jnp.float32),
            pltpu.SemaphoreType.DMA,
            pltpu.SemaphoreType.DMA,
            pltpu.SemaphoreType.DMA,
            pltpu.SemaphoreType.DMA,
        ],
        compiler_params=pltpu.CompilerParams(use_tc_tiling_on_sc=False),
    )(_body)
    return f(x, embeddings)


def kernel(x, embeddings):
    return _gather(x, embeddings)
